# baseline (speedup 1.0000x reference)
import functools

import jax
import jax.numpy as jnp
from jax import lax
from jax.experimental import pallas as pl
from jax.experimental.pallas import tpu as pltpu
from jax.experimental.pallas import tpu_sc as plsc

_LANES = 128
_G = 2
_CHUNK = _G * _LANES
_NBUF = 2


@functools.partial(jax.jit, static_argnames=("n_pairs",))
def _gather_pairs(xe, xo, t2, n_pairs):
    info = plsc.get_sparse_core_info()
    nw = info.num_cores * info.num_subcores
    per_w = n_pairs // nw
    steps = per_w // _CHUNK
    mesh = plsc.VectorSubcoreMesh(core_axis_name="c", subcore_axis_name="s")

    @functools.partial(
        pl.kernel,
        mesh=mesh,
        out_type=jax.ShapeDtypeStruct((n_pairs, 2 * 64), jnp.float32),
        scratch_types=[
            pltpu.VMEM((_CHUNK,), jnp.int32),
            pltpu.VMEM((_CHUNK,), jnp.int32),
            pltpu.VMEM((_G, _LANES), jnp.int32),
            pltpu.VMEM((_NBUF, _CHUNK, 2 * 64), jnp.float32),
            pltpu.SemaphoreType.DMA,
            pltpu.SemaphoreType.DMA,
            pltpu.SemaphoreType.DMA,
        ],
    )
    def k(xe_hbm, xo_hbm, t2_hbm, out_hbm, xe_v, xo_v, pair_v, rows_v,
          sem_g, sem_w0, sem_w1):
        wid = lax.axis_index("s") * info.num_cores + lax.axis_index("c")
        sem_w = (sem_w0, sem_w1)

        def outer(io, carry):
            for b in range(_NBUF):
                i = _NBUF * io + b
                p0 = wid * per_w + i * _CHUNK
                pltpu.sync_copy(xe_hbm.at[pl.ds(p0, _CHUNK)], xe_v)
                pltpu.sync_copy(xo_hbm.at[pl.ds(p0, _CHUNK)], xo_v)
                for t in range(_CHUNK // 16):
                    e = xe_v[pl.ds(t * 16, 16)]
                    o = xo_v[pl.ds(t * 16, 16)]
                    pair_v[t // 8, pl.ds((t % 8) * 16, 16)] = e * 10 + o

                @pl.when(io >= 1)
                def _():
                    pltpu.make_async_copy(
                        rows_v.at[b], out_hbm.at[pl.ds(0, _CHUNK)], sem_w[b]
                    ).wait()

                copies = [
                    pltpu.async_copy(
                        t2_hbm.at[pair_v.at[j]],
                        rows_v.at[b, pl.ds(j * _LANES, _LANES)],
                        sem_g,
                    )
                    for j in range(_G)
                ]
                for c in copies:
                    c.wait()
                pltpu.async_copy(
                    rows_v.at[b], out_hbm.at[pl.ds(p0, _CHUNK)], sem_w[b]
                )
            return carry

        lax.fori_loop(0, steps // _NBUF, outer, 0)
        for b in range(_NBUF):
            pltpu.make_async_copy(
                rows_v.at[b], out_hbm.at[pl.ds(0, _CHUNK)], sem_w[b]
            ).wait()

    return k(xe, xo, t2)


def kernel(x, table):
    b, s = x.shape
    v, d = table.shape
    n = b * s
    t2 = jnp.concatenate(
        [
            jnp.broadcast_to(table[:, None, :], (v, v, d)),
            jnp.broadcast_to(table[None, :, :], (v, v, d)),
        ],
        axis=-1,
    ).reshape(v * v, 2 * d)
    xp = x.reshape(n // 2, 2)
    out = _gather_pairs(xp[:, 0], xp[:, 1], t2, n // 2)
    return out.reshape(b, s, d)


# speedup vs baseline: 1.3434x; 1.3434x over previous
import functools

import jax
import jax.numpy as jnp
from jax import lax
from jax.experimental import pallas as pl
from jax.experimental.pallas import tpu as pltpu
from jax.experimental.pallas import tpu_sc as plsc

_LANES = 128
_G = 2
_CHUNK = _G * _LANES
_NBUF = 2


@functools.partial(jax.jit, static_argnames=("n_pairs",))
def _gather_pairs(xe, xo, t2, n_pairs):
    info = plsc.get_sparse_core_info()
    nw = info.num_cores * info.num_subcores
    per_w = n_pairs // nw
    steps = per_w // _CHUNK
    mesh = plsc.VectorSubcoreMesh(core_axis_name="c", subcore_axis_name="s")

    @functools.partial(
        pl.kernel,
        mesh=mesh,
        out_type=jax.ShapeDtypeStruct((n_pairs, 2 * 64), jnp.float32),
        scratch_types=[
            pltpu.VMEM((_CHUNK,), jnp.int32),
            pltpu.VMEM((_CHUNK,), jnp.int32),
            pltpu.VMEM((_G, _LANES), jnp.int32),
            pltpu.VMEM((_NBUF, _CHUNK, 2 * 64), jnp.float32),
            pltpu.VMEM_SHARED((100, 2 * 64), jnp.float32),
            pltpu.SemaphoreType.DMA,
            pltpu.SemaphoreType.DMA,
            pltpu.SemaphoreType.DMA,
        ],
    )
    def k(xe_hbm, xo_hbm, t2_hbm, out_hbm, xe_v, xo_v, pair_v, rows_v,
          t2_sh, sem_g, sem_w0, sem_w1):
        sid = lax.axis_index("s")
        wid = sid * info.num_cores + lax.axis_index("c")
        sem_w = (sem_w0, sem_w1)

        @pl.when(sid == 0)
        def _():
            pltpu.sync_copy(t2_hbm, t2_sh)

        plsc.subcore_barrier()

        def outer(io, carry):
            for b in range(_NBUF):
                i = _NBUF * io + b
                p0 = wid * per_w + i * _CHUNK
                pltpu.sync_copy(xe_hbm.at[pl.ds(p0, _CHUNK)], xe_v)
                pltpu.sync_copy(xo_hbm.at[pl.ds(p0, _CHUNK)], xo_v)
                for t in range(_CHUNK // 16):
                    e = xe_v[pl.ds(t * 16, 16)]
                    o = xo_v[pl.ds(t * 16, 16)]
                    pair_v[t // 8, pl.ds((t % 8) * 16, 16)] = e * 10 + o

                @pl.when(io >= 1)
                def _():
                    pltpu.make_async_copy(
                        rows_v.at[b], out_hbm.at[pl.ds(0, _CHUNK)], sem_w[b]
                    ).wait()

                copies = [
                    pltpu.async_copy(
                        t2_sh.at[pair_v.at[j]],
                        rows_v.at[b, pl.ds(j * _LANES, _LANES)],
                        sem_g,
                    )
                    for j in range(_G)
                ]
                for c in copies:
                    c.wait()
                pltpu.async_copy(
                    rows_v.at[b], out_hbm.at[pl.ds(p0, _CHUNK)], sem_w[b]
                )
            return carry

        lax.fori_loop(0, steps // _NBUF, outer, 0)
        for b in range(_NBUF):
            pltpu.make_async_copy(
                rows_v.at[b], out_hbm.at[pl.ds(0, _CHUNK)], sem_w[b]
            ).wait()

    return k(xe, xo, t2)


def kernel(x, table):
    b, s = x.shape
    v, d = table.shape
    n = b * s
    t2 = jnp.concatenate(
        [
            jnp.broadcast_to(table[:, None, :], (v, v, d)),
            jnp.broadcast_to(table[None, :, :], (v, v, d)),
        ],
        axis=-1,
    ).reshape(v * v, 2 * d)
    xp = x.reshape(n // 2, 2)
    out = _gather_pairs(xp[:, 0], xp[:, 1], t2, n // 2)
    return out.reshape(b, s, d)


# speedup vs baseline: 1.3464x; 1.0022x over previous
import functools

import jax
import jax.numpy as jnp
from jax import lax
from jax.experimental import pallas as pl
from jax.experimental.pallas import tpu as pltpu
from jax.experimental.pallas import tpu_sc as plsc

_LANES = 128
_G = 4
_CHUNK = _G * _LANES
_NBUF = 1


@functools.partial(jax.jit, static_argnames=("n_pairs",))
def _gather_pairs(xe, xo, t2, n_pairs):
    info = plsc.get_sparse_core_info()
    nw = info.num_cores * info.num_subcores
    per_w = n_pairs // nw
    steps = per_w // _CHUNK
    mesh = plsc.VectorSubcoreMesh(core_axis_name="c", subcore_axis_name="s")

    @functools.partial(
        pl.kernel,
        mesh=mesh,
        out_type=jax.ShapeDtypeStruct((n_pairs, 2 * 64), jnp.float32),
        scratch_types=[
            pltpu.VMEM((_CHUNK,), jnp.int32),
            pltpu.VMEM((_CHUNK,), jnp.int32),
            pltpu.VMEM((_G, _LANES), jnp.int32),
            pltpu.VMEM((_NBUF, _CHUNK, 2 * 64), jnp.float32),
            pltpu.VMEM_SHARED((100, 2 * 64), jnp.float32),
            pltpu.SemaphoreType.DMA,
            pltpu.SemaphoreType.DMA,
        ],
    )
    def k(xe_hbm, xo_hbm, t2_hbm, out_hbm, xe_v, xo_v, pair_v, rows_v,
          t2_sh, sem_g, sem_w0):
        sid = lax.axis_index("s")
        wid = sid * info.num_cores + lax.axis_index("c")
        sem_w = (sem_w0,)

        @pl.when(sid == 0)
        def _():
            pltpu.sync_copy(t2_hbm, t2_sh)

        plsc.subcore_barrier()

        def outer(io, carry):
            for b in range(_NBUF):
                i = _NBUF * io + b
                p0 = wid * per_w + i * _CHUNK
                pltpu.sync_copy(xe_hbm.at[pl.ds(p0, _CHUNK)], xe_v)
                pltpu.sync_copy(xo_hbm.at[pl.ds(p0, _CHUNK)], xo_v)
                for t in range(_CHUNK // 16):
                    e = xe_v[pl.ds(t * 16, 16)]
                    o = xo_v[pl.ds(t * 16, 16)]
                    pair_v[t // 8, pl.ds((t % 8) * 16, 16)] = e * 10 + o

                @pl.when(io >= 1)
                def _():
                    pltpu.make_async_copy(
                        rows_v.at[b], out_hbm.at[pl.ds(0, _CHUNK)], sem_w[b]
                    ).wait()

                copies = [
                    pltpu.async_copy(
                        t2_sh.at[pair_v.at[j]],
                        rows_v.at[b, pl.ds(j * _LANES, _LANES)],
                        sem_g,
                    )
                    for j in range(_G)
                ]
                for c in copies:
                    c.wait()
                pltpu.async_copy(
                    rows_v.at[b], out_hbm.at[pl.ds(p0, _CHUNK)], sem_w[b]
                )
            return carry

        lax.fori_loop(0, steps // _NBUF, outer, 0)
        for b in range(_NBUF):
            pltpu.make_async_copy(
                rows_v.at[b], out_hbm.at[pl.ds(0, _CHUNK)], sem_w[b]
            ).wait()

    return k(xe, xo, t2)


def kernel(x, table):
    b, s = x.shape
    v, d = table.shape
    n = b * s
    t2 = jnp.concatenate(
        [
            jnp.broadcast_to(table[:, None, :], (v, v, d)),
            jnp.broadcast_to(table[None, :, :], (v, v, d)),
        ],
        axis=-1,
    ).reshape(v * v, 2 * d)
    xp = x.reshape(n // 2, 2)
    out = _gather_pairs(xp[:, 0], xp[:, 1], t2, n // 2)
    return out.reshape(b, s, d)
